# Initial kernel scaffold; baseline (speedup 1.0000x reference)
import jax
import jax.numpy as jnp
from jax.experimental import pallas as pl


def kernel(x, W1, b1, g1, be1, Wd, bd, Wt, bt):
    raise NotImplementedError("write your pallas kernel here")


# speedup vs baseline: 2.6438x; 2.6438x over previous
import numpy as np
import jax
import jax.numpy as jnp
from jax.experimental import pallas as pl

_INP = 128
_TOPN = 8
_SIZE = 56
_NPIX = _SIZE * _SIZE
_NCH = 6
_B = 16


def _edge_anchors_np(inp_size):
    size = 3.0
    scales = [2 ** (1.0 / 3.0), 2 ** (2.0 / 3.0)]
    ars = [0.667, 1.0, 1.5]
    H = inp_size
    W = inp_size
    oy = np.arange(0.5, 0.5 + H, 1.0).reshape(H, 1)
    ox = np.arange(0.5, 0.5 + W, 1.0).reshape(1, W)
    edges = []
    for sc in scales:
        for ar in ars:
            cm = np.zeros((H, W, 4), dtype=np.float32)
            cm[:, :, 0] = oy
            cm[:, :, 1] = ox
            cm[:, :, 2] = size * sc / float(ar) ** 0.5
            cm[:, :, 3] = size * sc * float(ar) ** 0.5
            em = np.concatenate(
                (cm[..., :2] - cm[..., 2:4] / 2.0, cm[..., :2] + cm[..., 2:4] / 2.0),
                axis=-1)
            edges.append(em.reshape(-1, 4))
    return np.concatenate(edges, axis=0)


_ANCH_F = (_edge_anchors_np(_SIZE) + 1).astype(np.int64).astype(np.float32)
_ANCH4 = np.ascontiguousarray(_ANCH_F.T).reshape(4, _NCH, _NPIX)
_AREAS = ((_ANCH_F[:, 2] - _ANCH_F[:, 0]) *
          (_ANCH_F[:, 3] - _ANCH_F[:, 1])).reshape(_NCH, _NPIX)


def _k1(x_ref, w1_ref, b1_ref, y_ref, st_ref):
    b = pl.program_id(0)
    y = (jnp.dot(w1_ref[...], x_ref[0], preferred_element_type=jnp.float32)
         + b1_ref[0][:, None])
    y_ref[0] = y
    st = jnp.stack([jnp.sum(y, axis=1), jnp.sum(y * y, axis=1)])

    @pl.when(b == 0)
    def _():
        st_ref[...] = st

    @pl.when(b != 0)
    def _():
        st_ref[...] = st_ref[...] + st


def _k2(y_ref, st_ref, g_ref, be_ref, wd_ref, bd_ref, wt_ref, bt_ref,
        an_ref, ar_ref, o_ref):
    n = jnp.float32(_B * _NPIX)
    mean = st_ref[0] / n
    var = st_ref[1] / n - mean * mean
    scale = g_ref[0] / jnp.sqrt(var + 1e-5)
    shift = be_ref[0] - mean * scale
    feat = jnp.maximum(y_ref[0] * scale[:, None] + shift[:, None], 0.0)

    zp = jnp.zeros((64, 57), jnp.float32)
    fp = jnp.concatenate([zp, feat, zp], axis=1)
    xpos = jax.lax.broadcasted_iota(jnp.int32, (1, _NPIX), 1) % _SIZE
    acc = jnp.zeros((32, _NPIX), jnp.float32) + bd_ref[0][:, None]
    for ky in range(3):
        for kx in range(3):
            off = (ky - 1) * _SIZE + (kx - 1)
            sh = jax.lax.slice(fp, (0, 57 + off), (64, 57 + off + _NPIX))
            if kx == 0:
                sh = sh * (xpos >= 1).astype(jnp.float32)
            elif kx == 2:
                sh = sh * (xpos <= _SIZE - 2).astype(jnp.float32)
            acc = acc + jnp.dot(wd_ref[ky * 3 + kx], sh,
                                preferred_element_type=jnp.float32)
    d = jnp.maximum(acc, 0.0)
    s = (jnp.dot(wt_ref[...], d, preferred_element_type=jnp.float32)
         + bt_ref[0][:, None])

    ay0, ax0, ay1, ax1 = an_ref[0], an_ref[1], an_ref[2], an_ref[3]
    areas = ar_ref[...]
    ii0 = jax.lax.broadcasted_iota(jnp.int32, (_NCH, _NPIX), 0)
    ii1 = jax.lax.broadcasted_iota(jnp.int32, (_NCH, _NPIX), 1)
    idx2 = ii0 * _NPIX + ii1
    valid = jnp.ones((_NCH, _NPIX), jnp.bool_)
    neg = jnp.float32(-jnp.inf)
    last = jnp.int32(0)
    boxes = []
    for _ in range(_TOPN):
        ms = jnp.where(valid, s, neg)
        m = jnp.max(ms)
        anyv = m > neg
        pick = jnp.max(jnp.where(ms == m, idx2, -1))
        pick = jnp.where(anyv, pick, last)
        sel = idx2 == pick
        self_ = sel.astype(jnp.float32)
        cy0 = jnp.sum(ay0 * self_)
        cx0 = jnp.sum(ax0 * self_)
        cy1 = jnp.sum(ay1 * self_)
        cx1 = jnp.sum(ax1 * self_)
        boxes.append((cy0, cx0, cy1, cx1))
        carea = (cy1 - cy0) * (cx1 - cx0)
        ly = jnp.minimum(ay1, cy1) - jnp.maximum(ay0, cy0)
        lx = jnp.minimum(ax1, cx1) - jnp.maximum(ax0, cx0)
        inter = jnp.where((ly < 0) | (lx < 0), 0.0, ly * lx)
        iou = inter / (areas + carea - inter)
        valid = valid & (iou < 0.25) & jnp.logical_not(sel)
        last = pick

    pp = jax.lax.broadcasted_iota(jnp.int32, (1, _NPIX), 1)
    rpos = (pp // _SIZE).astype(jnp.float32)
    cpos = (pp % _SIZE).astype(jnp.float32)
    wrows = []
    cnts = []
    for (cy0, cx0, cy1, cx1) in boxes:
        y0 = jnp.clip(cy0, 0.0, 57.0)
        x0 = jnp.clip(cx0, 0.0, 57.0)
        y1 = jnp.maximum(y0 + 1.0, jnp.minimum(cy1, 58.0))
        x1 = jnp.maximum(x0 + 1.0, jnp.minimum(cx1, 58.0))
        wy = ((rpos + 1.0 >= y0) & (rpos + 1.0 < y1)).astype(jnp.float32)
        wy = wy + jnp.where(rpos == 0.0, ((y0 <= 0.0) & (y1 > 0.0)).astype(jnp.float32), 0.0)
        wy = wy + jnp.where(rpos == 55.0, (y1 > 57.0).astype(jnp.float32), 0.0)
        wx = ((cpos + 1.0 >= x0) & (cpos + 1.0 < x1)).astype(jnp.float32)
        wx = wx + jnp.where(cpos == 0.0, ((x0 <= 0.0) & (x1 > 0.0)).astype(jnp.float32), 0.0)
        wx = wx + jnp.where(cpos == 55.0, (x1 > 57.0).astype(jnp.float32), 0.0)
        wrows.append(wy * wx)
        cnts.append(jnp.zeros((1, 1), jnp.float32) + (y1 - y0) * (x1 - x0))
    wcat = jnp.concatenate(wrows, axis=0)
    cnt8 = jnp.concatenate(cnts, axis=0)
    pooled = jax.lax.dot_general(wcat, feat, (((1,), (1,)), ((), ())),
                                 preferred_element_type=jnp.float32)
    o_ref[0] = pooled / cnt8


def kernel(x, W1, b1, g1, be1, Wd, bd, Wt, bt):
    B = x.shape[0]
    x2 = x.reshape(B, _INP, _NPIX)
    w1 = W1.reshape(64, _INP)
    y, st = pl.pallas_call(
        _k1,
        grid=(B,),
        in_specs=[
            pl.BlockSpec((1, _INP, _NPIX), lambda b: (b, 0, 0)),
            pl.BlockSpec((64, _INP), lambda b: (0, 0)),
            pl.BlockSpec((1, 64), lambda b: (0, 0)),
        ],
        out_specs=[
            pl.BlockSpec((1, 64, _NPIX), lambda b: (b, 0, 0)),
            pl.BlockSpec((2, 64), lambda b: (0, 0)),
        ],
        out_shape=[
            jax.ShapeDtypeStruct((B, 64, _NPIX), jnp.float32),
            jax.ShapeDtypeStruct((2, 64), jnp.float32),
        ],
    )(x2, w1, b1.reshape(1, 64))

    wd9 = jnp.transpose(Wd.reshape(32, 64, 9), (2, 0, 1))
    anch = jnp.asarray(_ANCH4)
    areas = jnp.asarray(_AREAS)
    out = pl.pallas_call(
        _k2,
        grid=(B,),
        in_specs=[
            pl.BlockSpec((1, 64, _NPIX), lambda b: (b, 0, 0)),
            pl.BlockSpec((2, 64), lambda b: (0, 0)),
            pl.BlockSpec((1, 64), lambda b: (0, 0)),
            pl.BlockSpec((1, 64), lambda b: (0, 0)),
            pl.BlockSpec((9, 32, 64), lambda b: (0, 0, 0)),
            pl.BlockSpec((1, 32), lambda b: (0, 0)),
            pl.BlockSpec((6, 32), lambda b: (0, 0)),
            pl.BlockSpec((1, 6), lambda b: (0, 0)),
            pl.BlockSpec((4, _NCH, _NPIX), lambda b: (0, 0, 0)),
            pl.BlockSpec((_NCH, _NPIX), lambda b: (0, 0)),
        ],
        out_specs=pl.BlockSpec((1, _TOPN, 64), lambda b: (b, 0, 0)),
        out_shape=jax.ShapeDtypeStruct((B, _TOPN, 64), jnp.float32),
    )(y, st, g1.reshape(1, 64), be1.reshape(1, 64), wd9,
      bd.reshape(1, 32), Wt.reshape(6, 32), bt.reshape(1, 6),
      anch, areas)
    return out.reshape(B * _TOPN, 64, 1, 1)


# speedup vs baseline: 3.0921x; 1.1695x over previous
import numpy as np
import jax
import jax.numpy as jnp
from jax.experimental import pallas as pl
from jax.experimental.pallas import tpu as pltpu

_INP = 128
_TOPN = 8
_SIZE = 56
_NPIX = _SIZE * _SIZE
_NCH = 6
_B = 16
_NIMG = 4


def _edge_anchors_np(inp_size):
    size = 3.0
    scales = [2 ** (1.0 / 3.0), 2 ** (2.0 / 3.0)]
    ars = [0.667, 1.0, 1.5]
    H = inp_size
    W = inp_size
    oy = np.arange(0.5, 0.5 + H, 1.0).reshape(H, 1)
    ox = np.arange(0.5, 0.5 + W, 1.0).reshape(1, W)
    edges = []
    for sc in scales:
        for ar in ars:
            cm = np.zeros((H, W, 4), dtype=np.float32)
            cm[:, :, 0] = oy
            cm[:, :, 1] = ox
            cm[:, :, 2] = size * sc / float(ar) ** 0.5
            cm[:, :, 3] = size * sc * float(ar) ** 0.5
            em = np.concatenate(
                (cm[..., :2] - cm[..., 2:4] / 2.0, cm[..., :2] + cm[..., 2:4] / 2.0),
                axis=-1)
            edges.append(em.reshape(-1, 4))
    return np.concatenate(edges, axis=0)


_ANCH_F = (_edge_anchors_np(_SIZE) + 1).astype(np.int64).astype(np.float32)
_ANCH4 = np.ascontiguousarray(_ANCH_F.T).reshape(4, _NCH, _NPIX)
_AREAS = ((_ANCH_F[:, 2] - _ANCH_F[:, 0]) *
          (_ANCH_F[:, 3] - _ANCH_F[:, 1])).reshape(_NCH, _NPIX)

_HH = []
_WW = []
for _sc in [2 ** (1.0 / 3.0), 2 ** (2.0 / 3.0)]:
    for _ar in [0.667, 1.0, 1.5]:
        _h32 = np.float32(3.0 * _sc / float(_ar) ** 0.5)
        _w32 = np.float32(3.0 * _sc * float(_ar) ** 0.5)
        _HH.append(float(np.float32(_h32 / np.float32(2))))
        _WW.append(float(np.float32(_w32 / np.float32(2))))

_WIN = 960
_MARG = 285


def _sel6(ch, tbl):
    v = jnp.float32(tbl[0])
    for k in range(1, 6):
        v = jnp.where(ch == k, jnp.float32(tbl[k]), v)
    return v


def _ka(x_ref, g_ref, xs_ref):
    b = pl.program_id(0)
    xb = x_ref[0]
    G = jax.lax.dot_general(xb, xb, (((1,), (1,)), ((), ())),
                            preferred_element_type=jnp.float32)
    xs = jnp.sum(xb, axis=1, keepdims=True)

    @pl.when(b == 0)
    def _():
        g_ref[...] = G
        xs_ref[...] = xs

    @pl.when(b != 0)
    def _():
        g_ref[...] = g_ref[...] + G
        xs_ref[...] = xs_ref[...] + xs


def _kb(x_ref, g_ref, xs_ref, w1_ref, b1_ref, ga_ref, be_ref,
        wd_ref, bd_ref, wt_ref, bt_ref, an_ref, ar_ref, o_ref, ms_ref):
    n = jnp.float32(_B * _NPIX)
    w1 = w1_ref[...]
    mw = jnp.dot(w1, xs_ref[...], preferred_element_type=jnp.float32) / n
    s2 = jnp.sum(jnp.dot(w1, g_ref[...], preferred_element_type=jnp.float32) * w1,
                 axis=1, keepdims=True) / n
    var = s2 - mw * mw
    mean = mw + b1_ref[...]
    scale = ga_ref[...] / jnp.sqrt(var + 1e-5)
    shift = be_ref[...] - mean * scale

    zp = jnp.zeros((64, 57), jnp.float32)
    q = jax.lax.broadcasted_iota(jnp.int32, (1, 3250), 1) % _SIZE
    mlft = (q >= 1).astype(jnp.float32)
    mrgt = (q != 1).astype(jnp.float32)
    ii0 = jax.lax.broadcasted_iota(jnp.int32, (_NCH, _NPIX), 0)
    ii1 = jax.lax.broadcasted_iota(jnp.int32, (_NCH, _NPIX), 1)
    idx2 = ii0 * _NPIX + ii1
    iw0 = jax.lax.broadcasted_iota(jnp.int32, (_NCH, _WIN), 0)
    iw1 = jax.lax.broadcasted_iota(jnp.int32, (_NCH, _WIN), 1)
    pp = jax.lax.broadcasted_iota(jnp.int32, (_TOPN, _NPIX), 1)
    rpf = (pp // _SIZE).astype(jnp.float32)
    cpf = (pp % _SIZE).astype(jnp.float32)
    neg = jnp.float32(-jnp.inf)

    for i in range(_NIMG):
        y = (jnp.dot(w1, x_ref[i], preferred_element_type=jnp.float32)
             + b1_ref[...])
        feat = jnp.maximum(y * scale + shift, 0.0)

        fp = jnp.concatenate([zp, feat, zp], axis=1)
        fpL = fp * mlft
        fpR = fp * mrgt
        acc = jnp.zeros((32, _NPIX), jnp.float32) + bd_ref[...]
        for ky in range(3):
            for kx in range(3):
                off = (ky - 1) * _SIZE + (kx - 1)
                src = (fpL, fp, fpR)[kx]
                sh = jax.lax.slice(src, (0, 57 + off), (64, 57 + off + _NPIX))
                acc = acc + jnp.dot(wd_ref[ky * 3 + kx], sh,
                                    preferred_element_type=jnp.float32)
        d = jnp.maximum(acc, 0.0)
        s = (jnp.dot(wt_ref[...], d, preferred_element_type=jnp.float32)
             + bt_ref[...])

        ms_ref[i] = s
        last = jnp.int32(0)
        boxes = []
        for _ in range(_TOPN):
            ms = ms_ref[i]
            m = jnp.max(ms)
            anyv = m > neg
            pick = jnp.max(jnp.where(ms == m, idx2, -1))
            pick = jnp.where(anyv, pick, last)
            ch = pick // _NPIX
            rem = pick - ch * _NPIX
            gy = rem // _SIZE
            gx = rem - gy * _SIZE
            hh = _sel6(ch, _HH)
            ww = _sel6(ch, _WW)
            oyc = gy.astype(jnp.float32) + 0.5
            oxc = gx.astype(jnp.float32) + 0.5
            cy0 = jnp.trunc((oyc - hh) + 1.0)
            cx0 = jnp.trunc((oxc - ww) + 1.0)
            cy1 = jnp.trunc((oyc + hh) + 1.0)
            cx1 = jnp.trunc((oxc + ww) + 1.0)
            boxes.append((cy0, cx0, cy1, cx1))
            carea = (cy1 - cy0) * (cx1 - cx0)
            sp = jnp.minimum((jnp.maximum(rem - _MARG, 0) // 128) * 128,
                             _NPIX - _WIN)
            ay0 = an_ref[0, :, pl.ds(sp, _WIN)]
            ax0 = an_ref[1, :, pl.ds(sp, _WIN)]
            ay1 = an_ref[2, :, pl.ds(sp, _WIN)]
            ax1 = an_ref[3, :, pl.ds(sp, _WIN)]
            areas = ar_ref[:, pl.ds(sp, _WIN)]
            ly = jnp.minimum(ay1, cy1) - jnp.maximum(ay0, cy0)
            lx = jnp.minimum(ax1, cx1) - jnp.maximum(ax0, cx0)
            inter = jnp.where((ly < 0) | (lx < 0), 0.0, ly * lx)
            iou = inter / (areas + carea - inter)
            supp = (iou >= 0.25) | ((iw0 == ch) & (iw1 + sp == rem))
            msw = ms_ref[i, :, pl.ds(sp, _WIN)]
            ms_ref[i, :, pl.ds(sp, _WIN)] = jnp.where(supp, neg, msw)
            last = pick

        clipped = []
        for (cy0, cx0, cy1, cx1) in boxes:
            y0 = jnp.clip(cy0, 0.0, 57.0)
            x0 = jnp.clip(cx0, 0.0, 57.0)
            y1 = jnp.maximum(y0 + 1.0, jnp.minimum(cy1, 58.0))
            x1 = jnp.maximum(x0 + 1.0, jnp.minimum(cx1, 58.0))
            clipped.append((y0, x0, y1, x1))

        def _col(k):
            return jnp.concatenate(
                [jnp.full((1, 1), c[k], jnp.float32) for c in clipped], axis=0)

        y0b, x0b, y1b, x1b = _col(0), _col(1), _col(2), _col(3)
        wy = ((rpf + 1.0 >= y0b) & (rpf + 1.0 < y1b)).astype(jnp.float32)
        wy = wy + jnp.where(rpf == 0.0, (y0b <= 0.0).astype(jnp.float32), 0.0)
        wy = wy + jnp.where(rpf == 55.0, (y1b > 57.0).astype(jnp.float32), 0.0)
        wx = ((cpf + 1.0 >= x0b) & (cpf + 1.0 < x1b)).astype(jnp.float32)
        wx = wx + jnp.where(cpf == 0.0, (x0b <= 0.0).astype(jnp.float32), 0.0)
        wx = wx + jnp.where(cpf == 55.0, (x1b > 57.0).astype(jnp.float32), 0.0)
        wcat = wy * wx
        cnt8 = (y1b - y0b) * (x1b - x0b)
        pooled = jax.lax.dot_general(wcat, feat, (((1,), (1,)), ((), ())),
                                     preferred_element_type=jnp.float32)
        o_ref[i] = pooled / cnt8


def kernel(x, W1, b1, g1, be1, Wd, bd, Wt, bt):
    B = x.shape[0]
    x2 = x.reshape(B, _INP, _NPIX)
    G, xs = pl.pallas_call(
        _ka,
        grid=(B,),
        in_specs=[pl.BlockSpec((1, _INP, _NPIX), lambda b: (b, 0, 0))],
        out_specs=[
            pl.BlockSpec((_INP, _INP), lambda b: (0, 0)),
            pl.BlockSpec((_INP, 1), lambda b: (0, 0)),
        ],
        out_shape=[
            jax.ShapeDtypeStruct((_INP, _INP), jnp.float32),
            jax.ShapeDtypeStruct((_INP, 1), jnp.float32),
        ],
    )(x2)

    wd9 = jnp.transpose(Wd.reshape(32, 64, 9), (2, 0, 1))
    anch = jnp.asarray(_ANCH4)
    areas = jnp.asarray(_AREAS)
    nsteps = B // _NIMG
    out = pl.pallas_call(
        _kb,
        grid=(nsteps,),
        in_specs=[
            pl.BlockSpec((_NIMG, _INP, _NPIX), lambda b: (b, 0, 0)),
            pl.BlockSpec((_INP, _INP), lambda b: (0, 0)),
            pl.BlockSpec((_INP, 1), lambda b: (0, 0)),
            pl.BlockSpec((64, _INP), lambda b: (0, 0)),
            pl.BlockSpec((64, 1), lambda b: (0, 0)),
            pl.BlockSpec((64, 1), lambda b: (0, 0)),
            pl.BlockSpec((64, 1), lambda b: (0, 0)),
            pl.BlockSpec((9, 32, 64), lambda b: (0, 0, 0)),
            pl.BlockSpec((32, 1), lambda b: (0, 0)),
            pl.BlockSpec((6, 32), lambda b: (0, 0)),
            pl.BlockSpec((6, 1), lambda b: (0, 0)),
            pl.BlockSpec((4, _NCH, _NPIX), lambda b: (0, 0, 0)),
            pl.BlockSpec((_NCH, _NPIX), lambda b: (0, 0)),
        ],
        out_specs=pl.BlockSpec((_NIMG, _TOPN, 64), lambda b: (b, 0, 0)),
        out_shape=jax.ShapeDtypeStruct((B, _TOPN, 64), jnp.float32),
        scratch_shapes=[pltpu.VMEM((_NIMG, _NCH, _NPIX), jnp.float32)],
    )(x2, G, xs, W1.reshape(64, _INP), b1.reshape(64, 1), g1.reshape(64, 1),
      be1.reshape(64, 1), wd9, bd.reshape(32, 1), Wt.reshape(6, 32),
      bt.reshape(6, 1), anch, areas)
    return out.reshape(B * _TOPN, 64, 1, 1)


# speedup vs baseline: 3.3120x; 1.0711x over previous
import numpy as np
import jax
import jax.numpy as jnp
from jax.experimental import pallas as pl
from jax.experimental.pallas import tpu as pltpu

_INP = 128
_TOPN = 8
_SIZE = 56
_NPIX = _SIZE * _SIZE
_NCH = 6
_B = 16
_NIMG = 4


def _edge_anchors_np(inp_size):
    size = 3.0
    scales = [2 ** (1.0 / 3.0), 2 ** (2.0 / 3.0)]
    ars = [0.667, 1.0, 1.5]
    H = inp_size
    W = inp_size
    oy = np.arange(0.5, 0.5 + H, 1.0).reshape(H, 1)
    ox = np.arange(0.5, 0.5 + W, 1.0).reshape(1, W)
    edges = []
    for sc in scales:
        for ar in ars:
            cm = np.zeros((H, W, 4), dtype=np.float32)
            cm[:, :, 0] = oy
            cm[:, :, 1] = ox
            cm[:, :, 2] = size * sc / float(ar) ** 0.5
            cm[:, :, 3] = size * sc * float(ar) ** 0.5
            em = np.concatenate(
                (cm[..., :2] - cm[..., 2:4] / 2.0, cm[..., :2] + cm[..., 2:4] / 2.0),
                axis=-1)
            edges.append(em.reshape(-1, 4))
    return np.concatenate(edges, axis=0)


_ANCH_F = (_edge_anchors_np(_SIZE) + 1).astype(np.int64).astype(np.float32)
_ANCH4 = np.ascontiguousarray(_ANCH_F.T).reshape(4, _NCH, _NPIX)
_AREAS = ((_ANCH_F[:, 2] - _ANCH_F[:, 0]) *
          (_ANCH_F[:, 3] - _ANCH_F[:, 1])).reshape(_NCH, _NPIX)

_HH = []
_WW = []
for _sc in [2 ** (1.0 / 3.0), 2 ** (2.0 / 3.0)]:
    for _ar in [0.667, 1.0, 1.5]:
        _h32 = np.float32(3.0 * _sc / float(_ar) ** 0.5)
        _w32 = np.float32(3.0 * _sc * float(_ar) ** 0.5)
        _HH.append(float(np.float32(_h32 / np.float32(2))))
        _WW.append(float(np.float32(_w32 / np.float32(2))))

_WIN = 960
_MARG = 285


def _sel6(ch, tbl):
    v = jnp.float32(tbl[0])
    for k in range(1, 6):
        v = jnp.where(ch == k, jnp.float32(tbl[k]), v)
    return v


def _ka(x_ref, w1_ref, b1_ref, y_ref, st_ref):
    b = pl.program_id(0)
    y = (jnp.dot(w1_ref[...], x_ref[0], preferred_element_type=jnp.float32)
         + b1_ref[...])
    y_ref[0] = y
    st = jnp.concatenate([jnp.sum(y, axis=1, keepdims=True),
                          jnp.sum(y * y, axis=1, keepdims=True)], axis=1)

    @pl.when(b == 0)
    def _():
        st_ref[...] = st

    @pl.when(b != 0)
    def _():
        st_ref[...] = st_ref[...] + st


def _kb(y_all_ref, st_ref, ga_ref, be_ref,
        wd_ref, bd_ref, wt_ref, bt_ref, an_ref, ar_ref, o_ref, *ms_refs):
    n = jnp.float32(_B * _NPIX)
    mean = st_ref[:, 0:1] / n
    var = st_ref[:, 1:2] / n - mean * mean
    ga = ga_ref[...]
    be = be_ref[...]
    rstd = jnp.sqrt(var + 1e-5)

    zp = jnp.zeros((64, 57), jnp.float32)
    q = jax.lax.broadcasted_iota(jnp.int32, (1, 3250), 1) % _SIZE
    mlft = (q >= 1).astype(jnp.float32)
    mrgt = (q != 1).astype(jnp.float32)
    ii0 = jax.lax.broadcasted_iota(jnp.int32, (_NCH, _NPIX), 0)
    ii1 = jax.lax.broadcasted_iota(jnp.int32, (_NCH, _NPIX), 1)
    idx2 = ii0 * _NPIX + ii1
    iw0 = jax.lax.broadcasted_iota(jnp.int32, (_NCH, _WIN), 0)
    iw1 = jax.lax.broadcasted_iota(jnp.int32, (_NCH, _WIN), 1)
    pp = jax.lax.broadcasted_iota(jnp.int32, (_TOPN, _NPIX), 1)
    rpf = (pp // _SIZE).astype(jnp.float32)
    cpf = (pp % _SIZE).astype(jnp.float32)
    neg = jnp.float32(-jnp.inf)

    for i in range(_NIMG):
        y = y_all_ref[i]
        feat = jnp.maximum(ga * (y - mean) / rstd + be, 0.0)

        fp = jnp.concatenate([zp, feat, zp], axis=1)
        fpL = fp * mlft
        fpR = fp * mrgt
        acc = jnp.zeros((32, _NPIX), jnp.float32) + bd_ref[...]
        for ky in range(3):
            for kx in range(3):
                off = (ky - 1) * _SIZE + (kx - 1)
                src = (fpL, fp, fpR)[kx]
                sh = jax.lax.slice(src, (0, 57 + off), (64, 57 + off + _NPIX))
                acc = acc + jnp.dot(wd_ref[ky * 3 + kx], sh,
                                    preferred_element_type=jnp.float32)
        d = jnp.maximum(acc, 0.0)
        s = (jnp.dot(wt_ref[...], d, preferred_element_type=jnp.float32)
             + bt_ref[...])

        ms_ref = ms_refs[i]
        ms_ref[...] = s
        last = jnp.int32(0)
        boxes = []
        for _ in range(_TOPN):
            ms = ms_ref[...]
            m = jnp.max(ms)
            anyv = m > neg
            pick = jnp.max(jnp.where(ms == m, idx2, -1))
            pick = jnp.where(anyv, pick, last)
            ch = pick // _NPIX
            rem = pick - ch * _NPIX
            gy = rem // _SIZE
            gx = rem - gy * _SIZE
            hh = _sel6(ch, _HH)
            ww = _sel6(ch, _WW)
            oyc = gy.astype(jnp.float32) + 0.5
            oxc = gx.astype(jnp.float32) + 0.5
            cy0 = jnp.trunc((oyc - hh) + 1.0)
            cx0 = jnp.trunc((oxc - ww) + 1.0)
            cy1 = jnp.trunc((oyc + hh) + 1.0)
            cx1 = jnp.trunc((oxc + ww) + 1.0)
            boxes.append((cy0, cx0, cy1, cx1))
            carea = (cy1 - cy0) * (cx1 - cx0)
            sp = jnp.minimum((jnp.maximum(rem - _MARG, 0) // 128) * 128,
                             _NPIX - _WIN)
            ay0 = an_ref[0, :, pl.ds(sp, _WIN)]
            ax0 = an_ref[1, :, pl.ds(sp, _WIN)]
            ay1 = an_ref[2, :, pl.ds(sp, _WIN)]
            ax1 = an_ref[3, :, pl.ds(sp, _WIN)]
            areas = ar_ref[:, pl.ds(sp, _WIN)]
            ly = jnp.minimum(ay1, cy1) - jnp.maximum(ay0, cy0)
            lx = jnp.minimum(ax1, cx1) - jnp.maximum(ax0, cx0)
            inter = jnp.where((ly < 0) | (lx < 0), 0.0, ly * lx)
            supp = ((4.0 * inter >= areas + carea - inter)
                    | ((iw0 == ch) & (iw1 + sp == rem)))
            msw = ms_ref[:, pl.ds(sp, _WIN)]
            ms_ref[:, pl.ds(sp, _WIN)] = jnp.where(supp, neg, msw)
            last = pick

        clipped = []
        for (cy0, cx0, cy1, cx1) in boxes:
            y0 = jnp.clip(cy0, 0.0, 57.0)
            x0 = jnp.clip(cx0, 0.0, 57.0)
            y1 = jnp.maximum(y0 + 1.0, jnp.minimum(cy1, 58.0))
            x1 = jnp.maximum(x0 + 1.0, jnp.minimum(cx1, 58.0))
            clipped.append((y0, x0, y1, x1))

        def _col(k):
            return jnp.concatenate(
                [jnp.full((1, 1), c[k], jnp.float32) for c in clipped], axis=0)

        y0b, x0b, y1b, x1b = _col(0), _col(1), _col(2), _col(3)
        wy = ((rpf + 1.0 >= y0b) & (rpf + 1.0 < y1b)).astype(jnp.float32)
        wy = wy + jnp.where(rpf == 0.0, (y0b <= 0.0).astype(jnp.float32), 0.0)
        wy = wy + jnp.where(rpf == 55.0, (y1b > 57.0).astype(jnp.float32), 0.0)
        wx = ((cpf + 1.0 >= x0b) & (cpf + 1.0 < x1b)).astype(jnp.float32)
        wx = wx + jnp.where(cpf == 0.0, (x0b <= 0.0).astype(jnp.float32), 0.0)
        wx = wx + jnp.where(cpf == 55.0, (x1b > 57.0).astype(jnp.float32), 0.0)
        wcat = wy * wx
        cnt8 = (y1b - y0b) * (x1b - x0b)
        pooled = jax.lax.dot_general(wcat, feat, (((1,), (1,)), ((), ())),
                                     preferred_element_type=jnp.float32)
        o_ref[i] = pooled / cnt8


def kernel(x, W1, b1, g1, be1, Wd, bd, Wt, bt):
    B = x.shape[0]
    x2 = x.reshape(B, _INP, _NPIX)
    y, st = pl.pallas_call(
        _ka,
        grid=(B,),
        in_specs=[
            pl.BlockSpec((1, _INP, _NPIX), lambda b: (b, 0, 0)),
            pl.BlockSpec((64, _INP), lambda b: (0, 0)),
            pl.BlockSpec((64, 1), lambda b: (0, 0)),
        ],
        out_specs=[
            pl.BlockSpec((1, 64, _NPIX), lambda b: (b, 0, 0)),
            pl.BlockSpec((64, 2), lambda b: (0, 0)),
        ],
        out_shape=[
            jax.ShapeDtypeStruct((B, 64, _NPIX), jnp.float32),
            jax.ShapeDtypeStruct((64, 2), jnp.float32),
        ],
    )(x2, W1.reshape(64, _INP), b1.reshape(64, 1))

    wd9 = jnp.transpose(Wd.reshape(32, 64, 9), (2, 0, 1))
    anch = jnp.asarray(_ANCH4)
    areas = jnp.asarray(_AREAS)
    nsteps = B // _NIMG
    out = pl.pallas_call(
        _kb,
        grid=(nsteps,),
        in_specs=[
            pl.BlockSpec((_NIMG, 64, _NPIX), lambda b: (b, 0, 0)),
            pl.BlockSpec((64, 2), lambda b: (0, 0)),
            pl.BlockSpec((64, 1), lambda b: (0, 0)),
            pl.BlockSpec((64, 1), lambda b: (0, 0)),
            pl.BlockSpec((9, 32, 64), lambda b: (0, 0, 0)),
            pl.BlockSpec((32, 1), lambda b: (0, 0)),
            pl.BlockSpec((6, 32), lambda b: (0, 0)),
            pl.BlockSpec((6, 1), lambda b: (0, 0)),
            pl.BlockSpec((4, _NCH, _NPIX), lambda b: (0, 0, 0)),
            pl.BlockSpec((_NCH, _NPIX), lambda b: (0, 0)),
        ],
        out_specs=pl.BlockSpec((_NIMG, _TOPN, 64), lambda b: (b, 0, 0)),
        out_shape=jax.ShapeDtypeStruct((B, _TOPN, 64), jnp.float32),
        scratch_shapes=[pltpu.VMEM((_NCH, _NPIX), jnp.float32)
                        for _ in range(_NIMG)],
    )(y, st, g1.reshape(64, 1), be1.reshape(64, 1), wd9, bd.reshape(32, 1),
      Wt.reshape(6, 32), bt.reshape(6, 1), anch, areas)
    return out.reshape(B * _TOPN, 64, 1, 1)
